# baseline (speedup 1.0000x reference)
import jax
import jax.numpy as jnp
from jax import lax
from jax.experimental import pallas as pl
from jax.experimental.pallas import tpu as pltpu
from jax.experimental.pallas import tpu_sc as plsc

_NC = 2
_NS = 16
_NW = _NC * _NS

_DIM = 64
_IDX_CHUNK = 128
_CHUNK = 512
_GPS = _CHUNK // _IDX_CHUNK


def _make_emb(b_total: int):
    b_per_w = b_total // _NW
    steps = b_per_w // _CHUNK
    assert b_per_w * _NW == b_total and steps * _CHUNK == b_per_w

    mesh = plsc.VectorSubcoreMesh(core_axis_name="c", subcore_axis_name="s")

    def body(idx_hbm, table_hbm, out_hbm, idx_v, rows_v, sem_g):
        wid = lax.axis_index("s") * _NC + lax.axis_index("c")
        base = wid * b_per_w
        pltpu.sync_copy(idx_hbm.at[pl.ds(base, b_per_w)], idx_v)

        @pl.loop(0, steps)
        def _step(j):
            descs = []
            for g in range(_GPS):
                off = j * _CHUNK + g * _IDX_CHUNK
                descs.append(
                    pltpu.async_copy(
                        table_hbm.at[idx_v.at[pl.ds(off, _IDX_CHUNK)]],
                        rows_v.at[pl.ds(g * _IDX_CHUNK, _IDX_CHUNK)],
                        sem_g,
                    )
                )
            for d in descs:
                d.wait()
            pltpu.sync_copy(rows_v, out_hbm.at[pl.ds(base + j * _CHUNK, _CHUNK)])

    return pl.kernel(
        body,
        out_type=jax.ShapeDtypeStruct((b_total, _DIM), jnp.float32),
        mesh=mesh,
        compiler_params=pltpu.CompilerParams(use_tc_tiling_on_sc=False),
        scratch_types=[
            pltpu.VMEM((b_per_w,), jnp.int32),
            pltpu.VMEM((_CHUNK, _DIM), jnp.float32),
            pltpu.SemaphoreType.DMA,
        ],
    )


def kernel(token_ids, weight):
    lead_shape = token_ids.shape
    flat = token_ids.reshape(-1).astype(jnp.int32)
    out = _make_emb(flat.shape[0])(flat, weight)
    return out.reshape(*lead_shape, _DIM)


# speedup vs baseline: 1.0131x; 1.0131x over previous
import jax
import jax.numpy as jnp
from jax import lax
from jax.experimental import pallas as pl
from jax.experimental.pallas import tpu as pltpu
from jax.experimental.pallas import tpu_sc as plsc

_NC = 2
_NS = 16
_NW = _NC * _NS

_DIM = 64
_IDX_CHUNK = 1024
_CHUNK = 1024
_GPS = _CHUNK // _IDX_CHUNK


def _make_emb(b_total: int):
    b_per_w = b_total // _NW
    steps = b_per_w // _CHUNK
    assert b_per_w * _NW == b_total and steps * _CHUNK == b_per_w

    mesh = plsc.VectorSubcoreMesh(core_axis_name="c", subcore_axis_name="s")

    def body(idx_hbm, table_hbm, out_hbm, idx_v, rows_v, sem_g):
        wid = lax.axis_index("s") * _NC + lax.axis_index("c")
        base = wid * b_per_w
        pltpu.sync_copy(idx_hbm.at[pl.ds(base, b_per_w)], idx_v)

        @pl.loop(0, steps)
        def _step(j):
            descs = []
            for g in range(_GPS):
                off = j * _CHUNK + g * _IDX_CHUNK
                descs.append(
                    pltpu.async_copy(
                        table_hbm.at[idx_v.at[pl.ds(off, _IDX_CHUNK)]],
                        rows_v.at[pl.ds(g * _IDX_CHUNK, _IDX_CHUNK)],
                        sem_g,
                    )
                )
            for d in descs:
                d.wait()
            pltpu.sync_copy(rows_v, out_hbm.at[pl.ds(base + j * _CHUNK, _CHUNK)])

    return pl.kernel(
        body,
        out_type=jax.ShapeDtypeStruct((b_total, _DIM), jnp.float32),
        mesh=mesh,
        compiler_params=pltpu.CompilerParams(use_tc_tiling_on_sc=False),
        scratch_types=[
            pltpu.VMEM((b_per_w,), jnp.int32),
            pltpu.VMEM((_CHUNK, _DIM), jnp.float32),
            pltpu.SemaphoreType.DMA,
        ],
    )


def kernel(token_ids, weight):
    lead_shape = token_ids.shape
    flat = token_ids.reshape(-1).astype(jnp.int32)
    out = _make_emb(flat.shape[0])(flat, weight)
    return out.reshape(*lead_shape, _DIM)
